# Initial kernel scaffold; baseline (speedup 1.0000x reference)
import jax
import jax.numpy as jnp
from jax.experimental import pallas as pl


def kernel(inputs, tables, cardinality):
    raise NotImplementedError("write your pallas kernel here")


# speedup vs baseline: 1.0877x; 1.0877x over previous
import functools

import jax
import jax.numpy as jnp
from jax import lax
from jax.experimental import pallas as pl
from jax.experimental.pallas import tpu as pltpu
from jax.experimental.pallas import tpu_sc as plsc

B = 16384
F = 26
V = 100000
D = 32

NC = 2
NS = 16
L = 16
NW = NC * NS
RPW = B // NW
CHUNK = 128
NCHUNK = RPW // CHUNK


def _body(tbl_ref, inp_ref, cardb_ref, *refs):
    outs = refs[:F]
    idx_v, gidx_v, rows_v, card_v, gsem = refs[F:]
    wid = lax.axis_index("s") * NC + lax.axis_index("c")
    base = wid * RPW
    pltpu.sync_copy(cardb_ref, card_v)
    for f in range(F):
        pltpu.sync_copy(inp_ref.at[pl.ds(f * B + base, RPW)], idx_v)
        cvec = card_v[pl.ds(f * L, L)]

        def compute(i, _, cvec=cvec, foff=f * V):
            h = idx_v[pl.ds(i * L, L)] + 1
            h = jnp.where(h >= cvec, h - cvec, h)
            gidx_v[pl.ds(i * L, L)] = h + foff
            return 0

        lax.fori_loop(0, RPW // L, compute, 0)
        copies = [
            pltpu.async_copy(
                tbl_ref.at[gidx_v.at[pl.ds(j * CHUNK, CHUNK)]],
                rows_v.at[pl.ds(j * CHUNK, CHUNK)],
                gsem,
            )
            for j in range(NCHUNK)
        ]
        for c in copies:
            c.wait()
        pltpu.sync_copy(rows_v, outs[f].at[pl.ds(base, RPW)])


@jax.jit
def _run(tbl, inp_flat, card_b):
    mesh = plsc.VectorSubcoreMesh(core_axis_name="c", subcore_axis_name="s")
    fn = pl.kernel(
        _body,
        out_type=tuple(
            jax.ShapeDtypeStruct((B, D), jnp.float32) for _ in range(F)
        ),
        mesh=mesh,
        scratch_types=[
            pltpu.VMEM((RPW,), jnp.int32),
            pltpu.VMEM((RPW,), jnp.int32),
            pltpu.VMEM((RPW, D), jnp.float32),
            pltpu.VMEM((F * L,), jnp.int32),
            pltpu.SemaphoreType.DMA,
        ],
        compiler_params=pltpu.CompilerParams(use_tc_tiling_on_sc=False),
    )
    return fn(tbl, inp_flat, card_b)


def kernel(inputs, tables, cardinality):
    tbl = tables.reshape(F * V, D)
    inp_flat = inputs.T.reshape(F * B)
    card_b = jnp.broadcast_to(
        cardinality.astype(jnp.int32)[:, None], (F, L)
    ).reshape(F * L)
    return tuple(_run(tbl, inp_flat, card_b))


# speedup vs baseline: 1.1137x; 1.0239x over previous
import jax
import jax.numpy as jnp
from jax import lax
from jax.experimental import pallas as pl
from jax.experimental.pallas import tpu as pltpu
from jax.experimental.pallas import tpu_sc as plsc

B = 16384
F = 26
V = 100000
D = 32

NC = 2
NS = 16
L = 16
NW = NC * NS
RPW = B // NW
CHUNK = 128
NCHUNK = RPW // CHUNK
NBUF = 3


def _body(tbl_ref, inp_ref, cardb_ref, *refs):
    outs = refs[:F]
    idx_v, card_v = refs[F:F + 2]
    gidx = refs[F + 2:F + 2 + F]
    rows = refs[F + 2 + F:F + 2 + F + NBUF]
    gsems = refs[F + 2 + F + NBUF:F + 2 + F + 2 * NBUF]
    osems = refs[F + 2 + F + 2 * NBUF:]
    wid = lax.axis_index("s") * NC + lax.axis_index("c")
    base = wid * RPW

    pltpu.sync_copy(inp_ref.at[:, pl.ds(base, RPW)], idx_v)
    pltpu.sync_copy(cardb_ref, card_v)

    for f in range(F):
        cvec = card_v[pl.ds(f * L, L)]

        def compute(i, _, cvec=cvec, f=f):
            h = idx_v[f, pl.ds(i * L, L)] + 1
            h = jnp.where(h >= cvec, h - cvec, h)
            gidx[f][pl.ds(i * L, L)] = h + f * V
            return 0

        lax.fori_loop(0, RPW // L, compute, 0)

    gcopies = {}
    ocopies = {}
    for f in range(F + 1):
        if f < F:
            b = f % NBUF
            if f >= NBUF:
                ocopies.pop(f - NBUF).wait()
            gcopies[f] = [
                pltpu.async_copy(
                    tbl_ref.at[gidx[f].at[pl.ds(j * CHUNK, CHUNK)]],
                    rows[b].at[pl.ds(j * CHUNK, CHUNK)],
                    gsems[b],
                )
                for j in range(NCHUNK)
            ]
        if f >= 1:
            g = f - 1
            for c in gcopies.pop(g):
                c.wait()
            ocopies[g] = pltpu.async_copy(
                rows[g % NBUF], outs[g].at[pl.ds(base, RPW)], osems[g % NBUF]
            )
    for g in sorted(ocopies):
        ocopies.pop(g).wait()


@jax.jit
def _run(tbl, inp_t, card_b):
    mesh = plsc.VectorSubcoreMesh(core_axis_name="c", subcore_axis_name="s")
    fn = pl.kernel(
        _body,
        out_type=tuple(
            jax.ShapeDtypeStruct((B, D), jnp.float32) for _ in range(F)
        ),
        mesh=mesh,
        scratch_types=(
            [
                pltpu.VMEM((F, RPW), jnp.int32),
                pltpu.VMEM((F * L,), jnp.int32),
            ]
            + [pltpu.VMEM((RPW,), jnp.int32) for _ in range(F)]

            + [pltpu.VMEM((RPW, D), jnp.float32) for _ in range(NBUF)]
            + [pltpu.SemaphoreType.DMA for _ in range(2 * NBUF)]
        ),
        compiler_params=pltpu.CompilerParams(use_tc_tiling_on_sc=False),
    )
    return fn(tbl, inp_t, card_b)


def kernel(inputs, tables, cardinality):
    tbl = tables.reshape(F * V, D)
    inp_t = inputs.T
    card_b = jnp.broadcast_to(
        cardinality.astype(jnp.int32)[:, None], (F, L)
    ).reshape(F * L)
    return tuple(_run(tbl, inp_t, card_b))
